# baseline (speedup 1.0000x reference)
import functools

import jax
import jax.numpy as jnp
from jax import lax
from jax.experimental import pallas as pl
from jax.experimental.pallas import tpu as pltpu
from jax.experimental.pallas import tpu_sc as plsc

NC = 2
NS = 16
NW = NC * NS
CHUNK = 128
COMBO_ROWS = 3 * 3 * 23 * 8 * 7


def _build_tables_body(e1, e2, e3, e4, e5, i1, i2, i3, i4, i5, combo_ref, cidx_ref):
    a1, a2, a3, a4, a5 = e1[...], e2[...], e3[...], e4[...], e5[...]
    t = (a1[:, None, :] + a2[None, :, :]).reshape(9, 128)
    t = (t[:, None, :] + a3[None, :, :]).reshape(9 * 23, 128)
    t = (t[:, None, :] + a4[None, :, :]).reshape(9 * 23 * 8, 128)
    t = (t[:, None, :] + a5[None, :, :]).reshape(COMBO_ROWS, 128)
    combo_ref[...] = t
    cidx_ref[...] = (((i1[...] * 3 + i2[...]) * 23 + i3[...]) * 8 + i4[...]) * 7 + i5[...]


def _final_body(p_ref, x_ref, e1, e2, e3, e4, e5, w1, b1, w2, b2, out_ref):
    n = x_ref.shape[0]
    self_row = e1[0:1, :] + e2[0:1, :] + e3[0:1, :] + e4[0:1, :] + e5[0:1, :]
    aggr = p_ref[0, :n, :] + p_ref[1, :n, :] + x_ref[...] + self_row
    h = jnp.maximum(
        jnp.dot(aggr, w1[...], preferred_element_type=jnp.float32) + b1[...], 0.0)
    out_ref[...] = jnp.dot(h, w2[...], preferred_element_type=jnp.float32) + b2[...]


SKEW0 = 390.0 / 915.0


def _make_sc_kernel(n_nodes, e_pad, acc_rows):
    chunks_total = e_pad // (NS * CHUNK)
    nc0 = max(1, min(chunks_total - 1, int(chunks_total * SKEW0 + 0.5)))
    nc1 = chunks_total - nc0
    rows_per_sub = acc_rows // NS
    mesh = plsc.VectorSubcoreMesh(core_axis_name="c", subcore_axis_name="s")

    @functools.partial(
        pl.kernel,
        out_type=jax.ShapeDtypeStruct((NC, acc_rows, 128), jnp.float32),
        mesh=mesh,
        scratch_types=[
            pltpu.VMEM((CHUNK,), jnp.int32),
            pltpu.VMEM((CHUNK,), jnp.int32),
            pltpu.VMEM((CHUNK,), jnp.int32),
            pltpu.VMEM((CHUNK, 128), jnp.float32),
            pltpu.VMEM((CHUNK, 128), jnp.float32),
            pltpu.VMEM_SHARED((acc_rows, 128), jnp.float32),
            pltpu.SemaphoreType.DMA,
            pltpu.SemaphoreType.DMA,
        ],
    )
    def sc_kernel(x_hbm, combo_hbm, src_hbm, dst_hbm, cidx_hbm, zeros_hbm, out_hbm,
                  src_v, dst_v, cidx_v, xrows, crows, acc, sem1, sem2):
        cid = lax.axis_index("c")
        sid = lax.axis_index("s")
        wid = cid * NS + sid
        pltpu.sync_copy(zeros_hbm, acc.at[pl.ds(sid * rows_per_sub, rows_per_sub)])
        plsc.subcore_barrier()
        n_chunks = jnp.where(cid == 0, nc0, nc1)
        tile_base = jnp.where(cid == 0, sid * nc0,
                              NS * nc0 + sid * nc1) * CHUNK

        @pl.loop(0, n_chunks)
        def _(k):
            base = tile_base + k * CHUNK
            pltpu.sync_copy(src_hbm.at[pl.ds(base, CHUNK)], src_v)
            pltpu.sync_copy(dst_hbm.at[pl.ds(base, CHUNK)], dst_v)
            pltpu.sync_copy(cidx_hbm.at[pl.ds(base, CHUNK)], cidx_v)
            cp1 = pltpu.async_copy(x_hbm.at[src_v], xrows, sem1)
            cp2 = pltpu.async_copy(combo_hbm.at[cidx_v], crows, sem2)
            cp1.wait()
            cp2.wait()
            pltpu.sync_copy(xrows, acc.at[dst_v], add=True)
            pltpu.sync_copy(crows, acc.at[dst_v], add=True)

        plsc.subcore_barrier()
        pltpu.sync_copy(acc.at[pl.ds(sid * rows_per_sub, rows_per_sub)],
                        out_hbm.at[cid, pl.ds(sid * rows_per_sub, rows_per_sub)])

    return sc_kernel


def kernel(x, edge_index, is_conjugated, edge_is_aromatic, bond_type, bond_dir,
           bond_stereo, emb_conj, emb_arom, emb_btype, emb_bdir, emb_bstereo,
           W1, b1, W2, b2):
    n, d = x.shape
    e = edge_index.shape[1]
    e_pad = ((e + NW * CHUNK - 1) // (NW * CHUNK)) * (NW * CHUNK)
    acc_rows = ((n + 8 * NS - 1) // (8 * NS)) * (8 * NS)
    if acc_rows == n:
        acc_rows += 8 * NS
    pad = e_pad - e
    src = jnp.concatenate([edge_index[0], jnp.zeros((pad,), jnp.int32)])
    dst = jnp.concatenate([edge_index[1], jnp.full((pad,), n, jnp.int32)])

    def pad0(a):
        return jnp.concatenate([a, jnp.zeros((pad,), jnp.int32)]).reshape(e_pad // 128, 128)

    i1, i2, i3, i4, i5 = map(pad0, (is_conjugated, edge_is_aromatic, bond_type,
                                    bond_dir, bond_stereo))

    combo, cidx2d = pl.pallas_call(
        _build_tables_body,
        out_shape=[
            jax.ShapeDtypeStruct((COMBO_ROWS, 128), jnp.float32),
            jax.ShapeDtypeStruct((e_pad // 128, 128), jnp.int32),
        ],
    )(emb_conj, emb_arom, emb_btype, emb_bdir, emb_bstereo, i1, i2, i3, i4, i5)
    cidx = cidx2d.reshape(e_pad)

    zeros = jnp.zeros((acc_rows // NS, 128), jnp.float32)
    part = _make_sc_kernel(n, e_pad, acc_rows)(x, combo, src, dst, cidx, zeros)

    out = pl.pallas_call(
        _final_body,
        out_shape=jax.ShapeDtypeStruct((n, d), jnp.float32),
    )(part, x, emb_conj, emb_arom, emb_btype, emb_bdir, emb_bstereo,
      W1, b1.reshape(1, -1), W2, b2.reshape(1, -1))
    return out


# speedup vs baseline: 1.1058x; 1.1058x over previous
import functools

import jax
import jax.numpy as jnp
from jax import lax
from jax.experimental import pallas as pl
from jax.experimental.pallas import tpu as pltpu
from jax.experimental.pallas import tpu_sc as plsc

NC = 2
NS = 16
NW = NC * NS
CHUNK = 128
COMBO_ROWS = 3 * 3 * 23 * 8 * 7


def _build_tables_body(e1, e2, e3, e4, e5, i1, i2, i3, i4, i5, combo_ref, cidx_ref):
    a1, a2, a3, a4, a5 = e1[...], e2[...], e3[...], e4[...], e5[...]
    t = (a1[:, None, :] + a2[None, :, :]).reshape(9, 128)
    t = (t[:, None, :] + a3[None, :, :]).reshape(9 * 23, 128)
    t = (t[:, None, :] + a4[None, :, :]).reshape(9 * 23 * 8, 128)
    t = (t[:, None, :] + a5[None, :, :]).reshape(COMBO_ROWS, 128)
    combo_ref[...] = t
    cidx_ref[...] = (((i1[...] * 3 + i2[...]) * 23 + i3[...]) * 8 + i4[...]) * 7 + i5[...]


def _final_body(p_ref, x_ref, e1, e2, e3, e4, e5, w1, b1, w2, b2, out_ref):
    n = x_ref.shape[0]
    self_row = e1[0:1, :] + e2[0:1, :] + e3[0:1, :] + e4[0:1, :] + e5[0:1, :]
    aggr = p_ref[0, :n, :] + p_ref[1, :n, :] + x_ref[...] + self_row
    h = jnp.maximum(
        jnp.dot(aggr, w1[...], preferred_element_type=jnp.float32) + b1[...], 0.0)
    out_ref[...] = jnp.dot(h, w2[...], preferred_element_type=jnp.float32) + b2[...]


def _make_sc_kernel(n_nodes, e_pad, acc_rows):
    edges_per_tile = e_pad // NW
    n_chunks = edges_per_tile // CHUNK
    rows_per_sub = acc_rows // NS
    mesh = plsc.VectorSubcoreMesh(core_axis_name="c", subcore_axis_name="s")

    @functools.partial(
        pl.kernel,
        out_type=jax.ShapeDtypeStruct((NC, acc_rows, 128), jnp.float32),
        mesh=mesh,
        scratch_types=[
            pltpu.VMEM((CHUNK,), jnp.int32),
            pltpu.VMEM((CHUNK,), jnp.int32),
            pltpu.VMEM((CHUNK,), jnp.int32),
            pltpu.VMEM((CHUNK, 128), jnp.float32),
            pltpu.VMEM((CHUNK, 128), jnp.float32),
            pltpu.VMEM_SHARED((acc_rows, 128), jnp.float32),
            pltpu.SemaphoreType.DMA,
            pltpu.SemaphoreType.DMA,
        ],
    )
    def sc_kernel(x_hbm, combo_hbm, src_hbm, dst_hbm, cidx_hbm, out_hbm,
                  src_v, dst_v, cidx_v, xrows, crows, acc, sem1, sem2):
        cid = lax.axis_index("c")
        sid = lax.axis_index("s")
        wid = cid * NS + sid
        @pl.loop(0, CHUNK)
        def _(i):
            @pl.loop(0, 8)
            def _(j):
                xrows[i, pl.ds(j * 16, 16)] = jnp.zeros((16,), jnp.float32)

        zfull = rows_per_sub // CHUNK
        zrem = rows_per_sub % CHUNK
        zbase = sid * rows_per_sub

        @pl.loop(0, zfull)
        def _(r):
            pltpu.sync_copy(xrows, acc.at[pl.ds(zbase + r * CHUNK, CHUNK)])
        if zrem:
            pltpu.sync_copy(xrows.at[pl.ds(0, zrem)],
                            acc.at[pl.ds(zbase + zfull * CHUNK, zrem)])
        plsc.subcore_barrier()
        tile_base = wid * edges_per_tile

        @pl.loop(0, n_chunks)
        def _(k):
            base = tile_base + k * CHUNK
            pltpu.sync_copy(src_hbm.at[pl.ds(base, CHUNK)], src_v)
            pltpu.sync_copy(dst_hbm.at[pl.ds(base, CHUNK)], dst_v)
            pltpu.sync_copy(cidx_hbm.at[pl.ds(base, CHUNK)], cidx_v)
            cp1 = pltpu.async_copy(x_hbm.at[src_v], xrows, sem1)
            cp2 = pltpu.async_copy(combo_hbm.at[cidx_v], crows, sem2)
            cp1.wait()
            cp2.wait()
            pltpu.sync_copy(xrows, acc.at[dst_v], add=True)
            pltpu.sync_copy(crows, acc.at[dst_v], add=True)

        plsc.subcore_barrier()
        pltpu.sync_copy(acc.at[pl.ds(sid * rows_per_sub, rows_per_sub)],
                        out_hbm.at[cid, pl.ds(sid * rows_per_sub, rows_per_sub)])

    return sc_kernel


def kernel(x, edge_index, is_conjugated, edge_is_aromatic, bond_type, bond_dir,
           bond_stereo, emb_conj, emb_arom, emb_btype, emb_bdir, emb_bstereo,
           W1, b1, W2, b2):
    n, d = x.shape
    e = edge_index.shape[1]
    e_pad = ((e + NW * CHUNK - 1) // (NW * CHUNK)) * (NW * CHUNK)
    acc_rows = ((n + 8 * NS - 1) // (8 * NS)) * (8 * NS)
    if acc_rows == n:
        acc_rows += 8 * NS
    pad = e_pad - e
    src = jnp.concatenate([edge_index[0], jnp.zeros((pad,), jnp.int32)])
    dst = jnp.concatenate([edge_index[1], jnp.full((pad,), n, jnp.int32)])

    def pad0(a):
        return jnp.concatenate([a, jnp.zeros((pad,), jnp.int32)]).reshape(e_pad // 128, 128)

    i1, i2, i3, i4, i5 = map(pad0, (is_conjugated, edge_is_aromatic, bond_type,
                                    bond_dir, bond_stereo))

    combo, cidx2d = pl.pallas_call(
        _build_tables_body,
        out_shape=[
            jax.ShapeDtypeStruct((COMBO_ROWS, 128), jnp.float32),
            jax.ShapeDtypeStruct((e_pad // 128, 128), jnp.int32),
        ],
    )(emb_conj, emb_arom, emb_btype, emb_bdir, emb_bstereo, i1, i2, i3, i4, i5)
    cidx = cidx2d.reshape(e_pad)

    part = _make_sc_kernel(n, e_pad, acc_rows)(x, combo, src, dst, cidx)

    out = pl.pallas_call(
        _final_body,
        out_shape=jax.ShapeDtypeStruct((n, d), jnp.float32),
    )(part, x, emb_conj, emb_arom, emb_btype, emb_bdir, emb_bstereo,
      W1, b1.reshape(1, -1), W2, b2.reshape(1, -1))
    return out


# speedup vs baseline: 1.2209x; 1.1041x over previous
import functools

import jax
import jax.numpy as jnp
from jax import lax
from jax.experimental import pallas as pl
from jax.experimental.pallas import tpu as pltpu
from jax.experimental.pallas import tpu_sc as plsc

NC = 2
NS = 16
NW = NC * NS
CHUNK = 128
COMBO_ROWS = 3 * 3 * 23 * 8 * 7


def _build_tables_body(e1, e2, e3, e4, e5, i1, i2, i3, i4, i5, combo_ref, cidx_ref):
    a1, a2, a3, a4, a5 = e1[...], e2[...], e3[...], e4[...], e5[...]
    t = (a1[:, None, :] + a2[None, :, :]).reshape(9, 128)
    t = (t[:, None, :] + a3[None, :, :]).reshape(9 * 23, 128)
    t = (t[:, None, :] + a4[None, :, :]).reshape(9 * 23 * 8, 128)
    t = (t[:, None, :] + a5[None, :, :]).reshape(COMBO_ROWS, 128)
    combo_ref[...] = t
    cidx_ref[...] = (((i1[...] * 3 + i2[...]) * 23 + i3[...]) * 8 + i4[...]) * 7 + i5[...]


def _final_body(p_ref, x_ref, e1, e2, e3, e4, e5, w1, b1, w2, b2, out_ref):
    n = x_ref.shape[0]
    self_row = e1[0:1, :] + e2[0:1, :] + e3[0:1, :] + e4[0:1, :] + e5[0:1, :]
    aggr = p_ref[0, :n, :] + p_ref[1, :n, :] + x_ref[...] + self_row
    h = jnp.maximum(
        jnp.dot(aggr, w1[...], preferred_element_type=jnp.float32) + b1[...], 0.0)
    out_ref[...] = jnp.dot(h, w2[...], preferred_element_type=jnp.float32) + b2[...]


def _make_sc_kernel(n_nodes, e_pad, acc_rows):
    edges_per_tile = e_pad // NW
    n_chunks = edges_per_tile // CHUNK
    rows_per_sub = acc_rows // NS
    mesh = plsc.VectorSubcoreMesh(core_axis_name="c", subcore_axis_name="s")

    @functools.partial(
        pl.kernel,
        out_type=jax.ShapeDtypeStruct((NC, acc_rows, 128), jnp.float32),
        mesh=mesh,
        scratch_types=[
            pltpu.VMEM((3, CHUNK), jnp.int32),
            pltpu.VMEM((CHUNK, 128), jnp.float32),
            pltpu.VMEM((CHUNK, 128), jnp.float32),
            pltpu.VMEM_SHARED((acc_rows, 128), jnp.float32),
            pltpu.SemaphoreType.DMA,
            pltpu.SemaphoreType.DMA,
        ],
    )
    def sc_kernel(x_hbm, combo_hbm, idx_hbm, out_hbm,
                  idx_v, xrows, crows, acc, sem1, sem2):
        cid = lax.axis_index("c")
        sid = lax.axis_index("s")
        wid = cid * NS + sid
        @pl.loop(0, CHUNK)
        def _(i):
            @pl.loop(0, 8)
            def _(j):
                xrows[i, pl.ds(j * 16, 16)] = jnp.zeros((16,), jnp.float32)

        zfull = rows_per_sub // CHUNK
        zrem = rows_per_sub % CHUNK
        zbase = sid * rows_per_sub

        @pl.loop(0, zfull)
        def _(r):
            pltpu.sync_copy(xrows, acc.at[pl.ds(zbase + r * CHUNK, CHUNK)])
        if zrem:
            pltpu.sync_copy(xrows.at[pl.ds(0, zrem)],
                            acc.at[pl.ds(zbase + zfull * CHUNK, zrem)])
        plsc.subcore_barrier()
        tile_chunk_base = wid * n_chunks

        @pl.loop(0, n_chunks)
        def _(k):
            pltpu.sync_copy(idx_hbm.at[tile_chunk_base + k], idx_v)
            cp1 = pltpu.async_copy(x_hbm.at[idx_v.at[0]], xrows, sem1)
            cp2 = pltpu.async_copy(combo_hbm.at[idx_v.at[2]], crows, sem2)
            cp1.wait()
            cp2.wait()
            pltpu.sync_copy(xrows, acc.at[idx_v.at[1]], add=True)
            pltpu.sync_copy(crows, acc.at[idx_v.at[1]], add=True)

        plsc.subcore_barrier()
        pltpu.sync_copy(acc.at[pl.ds(sid * rows_per_sub, rows_per_sub)],
                        out_hbm.at[cid, pl.ds(sid * rows_per_sub, rows_per_sub)])

    return sc_kernel


def kernel(x, edge_index, is_conjugated, edge_is_aromatic, bond_type, bond_dir,
           bond_stereo, emb_conj, emb_arom, emb_btype, emb_bdir, emb_bstereo,
           W1, b1, W2, b2):
    n, d = x.shape
    e = edge_index.shape[1]
    e_pad = ((e + NW * CHUNK - 1) // (NW * CHUNK)) * (NW * CHUNK)
    acc_rows = ((n + 8 * NS - 1) // (8 * NS)) * (8 * NS)
    if acc_rows == n:
        acc_rows += 8 * NS
    pad = e_pad - e
    src = jnp.concatenate([edge_index[0], jnp.zeros((pad,), jnp.int32)])
    dst = jnp.concatenate([edge_index[1], jnp.full((pad,), n, jnp.int32)])

    def pad0(a):
        return jnp.concatenate([a, jnp.zeros((pad,), jnp.int32)]).reshape(e_pad // 128, 128)

    i1, i2, i3, i4, i5 = map(pad0, (is_conjugated, edge_is_aromatic, bond_type,
                                    bond_dir, bond_stereo))

    combo, cidx2d = pl.pallas_call(
        _build_tables_body,
        out_shape=[
            jax.ShapeDtypeStruct((COMBO_ROWS, 128), jnp.float32),
            jax.ShapeDtypeStruct((e_pad // 128, 128), jnp.int32),
        ],
    )(emb_conj, emb_arom, emb_btype, emb_bdir, emb_bstereo, i1, i2, i3, i4, i5)
    idx = jnp.stack([src.reshape(e_pad // CHUNK, CHUNK),
                     dst.reshape(e_pad // CHUNK, CHUNK),
                     cidx2d.reshape(e_pad // CHUNK, CHUNK)], axis=1)

    part = _make_sc_kernel(n, e_pad, acc_rows)(x, combo, idx)

    out = pl.pallas_call(
        _final_body,
        out_shape=jax.ShapeDtypeStruct((n, d), jnp.float32),
    )(part, x, emb_conj, emb_arom, emb_btype, emb_bdir, emb_bstereo,
      W1, b1.reshape(1, -1), W2, b2.reshape(1, -1))
    return out
